# baseline (speedup 1.0000x reference)
import functools

import jax
import jax.numpy as jnp
from jax import lax
from jax.experimental import pallas as pl
from jax.experimental.pallas import tpu as pltpu
from jax.experimental.pallas import tpu_sc as plsc

B, N, D = 4, 4096, 1024

NC, NS = 2, 16
NW = NC * NS
ROWS_PER_W = N // NW
CH = 32
NCHUNK = ROWS_PER_W // CH

_mesh = plsc.VectorSubcoreMesh(core_axis_name="c", subcore_axis_name="s")


@functools.partial(
    pl.kernel,
    out_type=jax.ShapeDtypeStruct((B, N, D), jnp.float32),
    mesh=_mesh,
    scratch_types=[
        pltpu.SemaphoreType.DMA,
    ],
)
def _pos_embed_sc(table_hbm, out_hbm, sem):
    wid = lax.axis_index("s") * NC + lax.axis_index("c")
    base = wid * ROWS_PER_W
    copies = [
        pltpu.async_copy(
            table_hbm.at[pl.ds(base, ROWS_PER_W)],
            out_hbm.at[b, pl.ds(base, ROWS_PER_W)],
            sem,
        )
        for b in range(B)
    ]
    for c in copies:
        c.wait()


def kernel(patches, table):
    del patches
    return _pos_embed_sc(table)


# speedup vs baseline: 39.3179x; 39.3179x over previous
import jax
import jax.numpy as jnp
from jax.experimental import pallas as pl

B, N, D = 4, 4096, 1024
BN = 512
NB = N // BN


def _body(table_ref, out_ref):
    out_ref[0] = table_ref[...]


def kernel(patches, table):
    del patches
    return pl.pallas_call(
        _body,
        grid=(NB, B),
        in_specs=[pl.BlockSpec((BN, D), lambda i, b: (i, 0))],
        out_specs=pl.BlockSpec((1, BN, D), lambda i, b: (b, i, 0)),
        out_shape=jax.ShapeDtypeStruct((B, N, D), jnp.float32),
    )(table[:N])


# speedup vs baseline: 44.8660x; 1.1411x over previous
import functools

import jax
import jax.numpy as jnp
from jax import lax
from jax.experimental import pallas as pl
from jax.experimental.pallas import tpu as pltpu
from jax.experimental.pallas import tpu_sc as plsc

B, N, D = 4, 4096, 1024

NC, NS = 2, 16
NW = NC * NS
ROWS_PER_W = N // NW
CH = 64
NCHUNK = ROWS_PER_W // CH

_mesh = plsc.VectorSubcoreMesh(core_axis_name="c", subcore_axis_name="s")


@functools.partial(
    pl.kernel,
    out_type=jax.ShapeDtypeStruct((B, N, D), jnp.float32),
    mesh=_mesh,
    scratch_types=[
        pltpu.VMEM((CH, D), jnp.float32),
        pltpu.SemaphoreType.DMA,
    ],
)
def _pos_embed_sc(table_hbm, out_hbm, buf, sem):
    wid = lax.axis_index("s") * NC + lax.axis_index("c")
    for g in range(NCHUNK):
        base = wid * ROWS_PER_W + g * CH
        pltpu.sync_copy(table_hbm.at[pl.ds(base, CH)], buf)
        copies = [
            pltpu.async_copy(buf, out_hbm.at[b, pl.ds(base, CH)], sem)
            for b in range(B)
        ]
        for c in copies:
            c.wait()


def kernel(patches, table):
    del patches
    return _pos_embed_sc(table)


# speedup vs baseline: 44.9389x; 1.0016x over previous
import functools

import jax
import jax.numpy as jnp
from jax import lax
from jax.experimental import pallas as pl
from jax.experimental.pallas import tpu as pltpu
from jax.experimental.pallas import tpu_sc as plsc

B, N, D = 4, 4096, 1024

NC, NS = 2, 16
NW = NC * NS
ROWS_PER_W = N // NW
CH = 64
NCHUNK = ROWS_PER_W // CH

_mesh = plsc.VectorSubcoreMesh(core_axis_name="c", subcore_axis_name="s")


@functools.partial(
    pl.kernel,
    out_type=jax.ShapeDtypeStruct((B, N, D), jnp.float32),
    mesh=_mesh,
    scratch_types=[
        pltpu.VMEM((CH, D), jnp.float32),
        pltpu.SemaphoreType.DMA,
    ],
)
def _pos_embed_sc(table_hbm, out_hbm, buf, sem):
    wid = lax.axis_index("c") * NS + lax.axis_index("s")
    for g in range(NCHUNK):
        base = wid * ROWS_PER_W + g * CH
        pltpu.sync_copy(table_hbm.at[pl.ds(base, CH)], buf)
        copies = [
            pltpu.async_copy(buf, out_hbm.at[b, pl.ds(base, CH)], sem)
            for b in range(B)
        ]
        for c in copies:
            c.wait()


def kernel(patches, table):
    del patches
    return _pos_embed_sc(table)


# speedup vs baseline: 45.2418x; 1.0067x over previous
import functools

import jax
import jax.numpy as jnp
from jax import lax
from jax.experimental import pallas as pl
from jax.experimental.pallas import tpu as pltpu
from jax.experimental.pallas import tpu_sc as plsc

B, N, D = 4, 4096, 1024

NC, NS = 2, 16
NW = NC * NS
ROWS_PER_W = N // NW
CH = 64
NCHUNK = ROWS_PER_W // CH

_mesh = plsc.VectorSubcoreMesh(core_axis_name="c", subcore_axis_name="s")


@functools.partial(
    pl.kernel,
    out_type=jax.ShapeDtypeStruct((B, N, D), jnp.float32),
    mesh=_mesh,
    scratch_types=[
        pltpu.VMEM((CH, D), jnp.float32),
        pltpu.SemaphoreType.DMA,
    ],
)
def _pos_embed_sc(table_hbm, out_hbm, buf, sem):
    wid = lax.axis_index("s") * NC + lax.axis_index("c")
    for g in range(NCHUNK):
        base = wid * ROWS_PER_W + g * CH
        pltpu.sync_copy(table_hbm.at[pl.ds(base, CH)], buf)
        copies = [
            pltpu.async_copy(buf, out_hbm.at[b, pl.ds(base, CH)], sem)
            for b in range(B)
        ]
        for c in copies:
            c.wait()


def kernel(patches, table):
    del patches
    return _pos_embed_sc(table)
